# baseline (speedup 1.0000x reference)
import functools

import jax
import jax.numpy as jnp
from jax import lax
from jax.experimental import pallas as pl
from jax.experimental.pallas import tpu as pltpu
from jax.experimental.pallas import tpu_sc as plsc

N = 10000
E = 160000
L = 64
CNN_C = 64
NEWS_P = 32
GCN_IN = CNN_C + NEWS_P
G_HID = 256
G_OUT = 128

NB = 400
NBLK = N // NB
ACC_N = 10112
SLAB = ACC_N // 16
EB = 128
EP = 16 * 80 * EB
T0 = 16
T1 = 80 - T0
TMAX = max(T0, T1)
CHW = 128
NBUF = 2


def _deg_body(dst_ref, out_ref):
    i = pl.program_id(0)
    d = dst_ref[0, 0, :]
    hi = d >> 7
    lo = d & 127
    oh_hi = (hi[:, None] == lax.broadcasted_iota(jnp.int32, (4096, 80), 1)
             ).astype(jnp.float32)
    oh_lo = (lo[:, None] == lax.broadcasted_iota(jnp.int32, (4096, 128), 1)
             ).astype(jnp.float32)
    contrib = lax.dot_general(oh_hi, oh_lo, (((0,), (0,)), ((), ())),
                              preferred_element_type=jnp.float32)

    @pl.when(i == 0)
    def _():
        out_ref[...] = jnp.zeros_like(out_ref)

    out_ref[...] += contrib


def _deg_hist(dst2d):
    return pl.pallas_call(
        _deg_body,
        grid=(dst2d.shape[0],),
        in_specs=[pl.BlockSpec((1, 1, 4096), lambda i: (i, 0, 0))],
        out_specs=pl.BlockSpec((80, 128), lambda i: (0, 0)),
        out_shape=jax.ShapeDtypeStruct((80, 128), jnp.float32),
    )(dst2d)


def _b1_body(xp_ref, news_ref, M_ref, brep_ref, w1_ref, b1_ref, w2_ref,
             b2_ref, out_ref):
    feats = []
    for b in range(2):
        xp = xp_ref[b]
        y = jnp.dot(xp, M_ref[...], preferred_element_type=jnp.float32)
        y = jnp.maximum(y + brep_ref[0][None, :], 0.0)
        cnn = jnp.mean(y.reshape(NB, CNN_C, L), axis=-1)
        nw = news_ref[b]
        h = jnp.dot(nw, w1_ref[...], preferred_element_type=jnp.float32)
        h = jnp.maximum(h + b1_ref[0][None, :], 0.0)
        nf = jnp.dot(h, w2_ref[...], preferred_element_type=jnp.float32)
        nf = nf + b2_ref[0][None, :]
        feats += [cnn, nf]
    out_ref[...] = jnp.concatenate(feats, axis=1)


def _fused_features(xpad, news, M, brep, npw1, npb1, npw2, npb2):
    return pl.pallas_call(
        _b1_body,
        grid=(NBLK,),
        in_specs=[
            pl.BlockSpec((2, NB, L + 2), lambda i: (0, i, 0)),
            pl.BlockSpec((2, NB, 128), lambda i: (0, i, 0)),
            pl.BlockSpec((L + 2, CNN_C * L), lambda i: (0, 0)),
            pl.BlockSpec((1, CNN_C * L), lambda i: (0, 0)),
            pl.BlockSpec((128, 64), lambda i: (0, 0)),
            pl.BlockSpec((1, 64), lambda i: (0, 0)),
            pl.BlockSpec((64, 32), lambda i: (0, 0)),
            pl.BlockSpec((1, 32), lambda i: (0, 0)),
        ],
        out_specs=pl.BlockSpec((NB, 2 * GCN_IN), lambda i: (i, 0)),
        out_shape=jax.ShapeDtypeStruct((N, 2 * GCN_IN), jnp.float32),
    )(xpad, news, M, brep, npw1, npb1, npw2, npb2)


def _b2_body(f2_ref, W_ref, bb_ref, deg_ref, g0, g1, g2, g3, dinv_ref):
    h = jnp.dot(f2_ref[...], W_ref[...], preferred_element_type=jnp.float32)
    h = h + bb_ref[0][None, :]
    deg = deg_ref[...] + 1.0
    dinv = lax.rsqrt(jnp.maximum(deg, 1.0))
    dinv_ref[...] = dinv
    g = h * dinv
    outs = (g0, g1, g2, g3)
    for c in range(4):
        outs[c][...] = g[:, c * CHW:(c + 1) * CHW]


def _g1_tables(f2, Wbig1, bbig1, deg_col):
    outs = tuple(jax.ShapeDtypeStruct((N, CHW), jnp.float32) for _ in range(4))
    return pl.pallas_call(
        _b2_body,
        grid=(NBLK,),
        in_specs=[
            pl.BlockSpec((NB, 2 * GCN_IN), lambda i: (i, 0)),
            pl.BlockSpec((2 * GCN_IN, 2 * G_HID), lambda i: (0, 0)),
            pl.BlockSpec((1, 2 * G_HID), lambda i: (0, 0)),
            pl.BlockSpec((NB, 1), lambda i: (i, 0)),
        ],
        out_specs=tuple(pl.BlockSpec((NB, CHW), lambda i: (i, 0))
                        for _ in range(4)) + (pl.BlockSpec((NB, 1), lambda i: (i, 0)),),
        out_shape=outs + (jax.ShapeDtypeStruct((N, 1), jnp.float32),),
    )(f2, Wbig1, bbig1, deg_col)


def _make_sc_scatter(nch):
    mesh = plsc.VectorSubcoreMesh(core_axis_name="c", subcore_axis_name="s")

    @functools.partial(
        pl.kernel,
        mesh=mesh,
        out_type=jax.ShapeDtypeStruct((nch, 2, ACC_N, CHW), jnp.float32),
        scratch_types=[
            pltpu.VMEM((TMAX, EB), jnp.int32),
            pltpu.VMEM((TMAX, EB), jnp.int32),
            pltpu.VMEM((NBUF, EB, CHW), jnp.float32),
            pltpu.VMEM_SHARED((ACC_N, CHW), jnp.float32),
            pltpu.SemaphoreType.DMA,
        ],
    )
    def sc_scatter(src_hbm, dst_hbm, zeros_hbm, *rest):
        tables = rest[:nch]
        out = rest[nch]
        src_v, dst_v, gb, acc, sem = rest[nch + 1:]
        c = lax.axis_index("c")
        s = lax.axis_index("s")

        def run(first, nt):
            pltpu.sync_copy(src_hbm.at[s, pl.ds(first, nt)],
                            src_v.at[pl.ds(0, nt)])
            pltpu.sync_copy(dst_hbm.at[s, pl.ds(first, nt)],
                            dst_v.at[pl.ds(0, nt)])
            for ch in range(nch):
                pltpu.sync_copy(zeros_hbm, acc.at[pl.ds(s * SLAB, SLAB)])
                plsc.subcore_barrier()

                for b in range(NBUF):
                    pltpu.async_copy(tables[ch].at[src_v.at[b]], gb.at[b], sem)

                def body(t, carry):
                    b = jnp.bitwise_and(t, NBUF - 1)
                    pltpu.make_async_copy(tables[ch].at[src_v.at[t]],
                                          gb.at[b], sem).wait()
                    pltpu.sync_copy(gb.at[b], acc.at[dst_v.at[t]], add=True)

                    @pl.when(t + NBUF < nt)
                    def _():
                        pltpu.async_copy(tables[ch].at[src_v.at[t + NBUF]],
                                         gb.at[b], sem)
                    return carry

                lax.fori_loop(0, nt, body, 0)
                plsc.subcore_barrier()
                pltpu.sync_copy(acc.at[pl.ds(s * SLAB, SLAB)],
                                out.at[ch, c, pl.ds(s * SLAB, SLAB)])

        @pl.when(c == 0)
        def _():
            run(0, T0)

        @pl.when(c == 1)
        def _():
            run(T0, T1)

    return sc_scatter


_sc_cache = {}


def _sc_scatter(nch, *args):
    if nch not in _sc_cache:
        _sc_cache[nch] = _make_sc_scatter(nch)
    return _sc_cache[nch](*args)


def _c_body(P_ref, g10, g11, g12, g13, dinv_ref, W_ref, bb_ref, o0, o1):
    dinv = dinv_ref[...]
    g1s = (g10, g11, g12, g13)
    xs = []
    for ch in range(4):
        S = P_ref[ch, 0] + P_ref[ch, 1]
        xs.append(jnp.maximum((S + g1s[ch][...]) * dinv, 0.0))
    x1 = jnp.concatenate(xs, axis=1)
    h = jnp.dot(x1, W_ref[...], preferred_element_type=jnp.float32)
    h = h + bb_ref[0][None, :]
    g = h * dinv
    o0[...] = g[:, :CHW]
    o1[...] = g[:, CHW:]


def _layer2_tables(P1, g1s, dinv_col, Wbig2, bbig2):
    outs = tuple(jax.ShapeDtypeStruct((N, CHW), jnp.float32) for _ in range(2))
    return pl.pallas_call(
        _c_body,
        grid=(NBLK,),
        in_specs=[
            pl.BlockSpec((4, 2, NB, CHW), lambda i: (0, 0, i, 0)),
            pl.BlockSpec((NB, CHW), lambda i: (i, 0)),
            pl.BlockSpec((NB, CHW), lambda i: (i, 0)),
            pl.BlockSpec((NB, CHW), lambda i: (i, 0)),
            pl.BlockSpec((NB, CHW), lambda i: (i, 0)),
            pl.BlockSpec((NB, 1), lambda i: (i, 0)),
            pl.BlockSpec((2 * G_HID, 2 * G_OUT), lambda i: (0, 0)),
            pl.BlockSpec((1, 2 * G_OUT), lambda i: (0, 0)),
        ],
        out_specs=tuple(pl.BlockSpec((NB, CHW), lambda i: (i, 0))
                        for _ in range(2)),
        out_shape=outs,
    )(P1, *g1s, dinv_col, Wbig2, bbig2)


def _f_body(P_ref, g20, g21, dinv_ref, W1_ref, b1_ref, W2_ref, b2_ref, o_ref):
    dinv = dinv_ref[...]
    g2s = (g20, g21)
    xs = []
    for ch in range(2):
        S = P_ref[ch, 0] + P_ref[ch, 1]
        xs.append((S + g2s[ch][...]) * dinv)
    x2 = jnp.concatenate(xs, axis=1)
    h = jnp.dot(x2, W1_ref[...], preferred_element_type=jnp.float32)
    h = jnp.maximum(h + b1_ref[0][None, :], 0.0)
    o = jnp.dot(h, W2_ref[...], preferred_element_type=jnp.float32)
    o_ref[...] = o + b2_ref[0][None, :]


def _final_mlp(P2, g2s, dinv_col, Mw1b, mb1b, Mw2b, mb2b):
    return pl.pallas_call(
        _f_body,
        grid=(NBLK,),
        in_specs=[
            pl.BlockSpec((2, 2, NB, CHW), lambda i: (0, 0, i, 0)),
            pl.BlockSpec((NB, CHW), lambda i: (i, 0)),
            pl.BlockSpec((NB, CHW), lambda i: (i, 0)),
            pl.BlockSpec((NB, 1), lambda i: (i, 0)),
            pl.BlockSpec((2 * G_OUT, 2 * G_OUT), lambda i: (0, 0)),
            pl.BlockSpec((1, 2 * G_OUT), lambda i: (0, 0)),
            pl.BlockSpec((2 * G_OUT, 4), lambda i: (0, 0)),
            pl.BlockSpec((1, 4), lambda i: (0, 0)),
        ],
        out_specs=pl.BlockSpec((NB, 4), lambda i: (i, 0)),
        out_shape=jax.ShapeDtypeStruct((N, 4), jnp.float32),
    )(P2, *g2s, dinv_col, Mw1b, mb1b, Mw2b, mb2b)


def _blockdiag2(w):
    r, c = w.shape
    out = jnp.zeros((2 * r, 2 * c), jnp.float32)
    out = out.at[:r, :c].set(w)
    out = out.at[r:, c:].set(w)
    return out


def kernel(price_data_x, edge_index, news_features, conv_w, conv_b,
           npw1, npb1, npw2, npb2, gw1, gb1, gw2, gb2, mw1, mb1, mw2, mb2):
    f32 = jnp.float32
    src = edge_index[0]
    dst = edge_index[1]

    npad = EP - E
    src_pad = jnp.concatenate([src, jnp.zeros((npad,), jnp.int32)])
    dst_pad = jnp.concatenate([dst, jnp.full((npad,), N, jnp.int32)])
    src_r = src_pad.reshape(16, 80, EB)
    dst_r = dst_pad.reshape(16, 80, EB)
    dst_deg = jnp.concatenate([dst, jnp.full((npad,), 10240, jnp.int32)])
    dst_deg2d = dst_deg.reshape(EP // 4096, 1, 4096)
    zeros_slab = jnp.zeros((SLAB, CHW), f32)

    j_i = lax.broadcasted_iota(jnp.int32, (L + 2, CNN_C, L), 0)
    t_i = lax.broadcasted_iota(jnp.int32, (L + 2, CNN_C, L), 2)
    d = j_i - t_i
    M = jnp.zeros((L + 2, CNN_C, L), f32)
    for k in range(3):
        M = M + jnp.where(d == k, conv_w[None, :, 0, k, None], 0.0)
    M = M.reshape(L + 2, CNN_C * L)
    brep = jnp.repeat(conv_b, L).reshape(1, CNN_C * L)

    Wbig1 = _blockdiag2(gw1)
    bbig1 = jnp.concatenate([gb1, gb1]).reshape(1, -1)
    Wbig2 = _blockdiag2(gw2)
    bbig2 = jnp.concatenate([gb2, gb2]).reshape(1, -1)
    Mw1b = _blockdiag2(mw1)
    mb1b = jnp.concatenate([mb1, mb1]).reshape(1, -1)
    Mw2b = _blockdiag2(mw2)
    mb2b = jnp.concatenate([mb2, mb2]).reshape(1, -1)

    xpad = jnp.pad(price_data_x, ((0, 0), (0, 0), (1, 1)))

    deg2d = _deg_hist(dst_deg2d)
    deg_col = deg2d.reshape(10240)[:N].reshape(N, 1)

    f2 = _fused_features(xpad, news_features, M, brep,
                         npw1, npb1.reshape(1, -1), npw2, npb2.reshape(1, -1))

    *g1s, dinv_col = _g1_tables(f2, Wbig1, bbig1, deg_col)

    P1 = _sc_scatter(4, src_r, dst_r, zeros_slab, *g1s)

    g2s = _layer2_tables(P1, g1s, dinv_col, Wbig2, bbig2)

    P2 = _sc_scatter(2, src_r, dst_r, zeros_slab, *g2s)

    out4 = _final_mlp(P2, g2s, dinv_col, Mw1b, mb1b, Mw2b, mb2b)
    return out4.reshape(N, 2, 2).transpose(1, 0, 2)


# speedup vs baseline: 1.3865x; 1.3865x over previous
import functools

import jax
import jax.numpy as jnp
from jax import lax
from jax.experimental import pallas as pl
from jax.experimental.pallas import tpu as pltpu
from jax.experimental.pallas import tpu_sc as plsc

N = 10000
E = 160000
L = 64
CNN_C = 64
NEWS_P = 32
GCN_IN = CNN_C + NEWS_P
G_HID = 256
G_OUT = 128

NB = 400
NBLK = N // NB
ACC_N = 10112
SLAB = ACC_N // 16
EB = 128
EP = 16 * 80 * EB
T0 = 64
T1 = 80 - T0
TMAX = max(T0, T1)
CHW = 128
NBUF = 2


def _deg_body(dst_ref, out_ref):
    i = pl.program_id(0)
    d = dst_ref[0, 0, :]
    hi = d >> 7
    lo = d & 127
    oh_hi = (hi[:, None] == lax.broadcasted_iota(jnp.int32, (4096, 80), 1)
             ).astype(jnp.bfloat16)
    oh_lo = (lo[:, None] == lax.broadcasted_iota(jnp.int32, (4096, 128), 1)
             ).astype(jnp.bfloat16)
    contrib = lax.dot_general(oh_hi, oh_lo, (((0,), (0,)), ((), ())),
                              preferred_element_type=jnp.float32)

    @pl.when(i == 0)
    def _():
        out_ref[...] = jnp.zeros_like(out_ref)

    out_ref[...] += contrib


def _deg_hist(dst2d):
    return pl.pallas_call(
        _deg_body,
        grid=(dst2d.shape[0],),
        in_specs=[pl.BlockSpec((1, 1, 4096), lambda i: (i, 0, 0))],
        out_specs=pl.BlockSpec((80, 128), lambda i: (0, 0)),
        out_shape=jax.ShapeDtypeStruct((80, 128), jnp.float32),
    )(dst2d)


def _b1_body(xp_ref, news_ref, M_ref, P_ref, w1_ref, b1_ref, w2_ref,
             b2_ref, out_ref):
    feats = []
    for b in range(2):
        xp = xp_ref[b].astype(jnp.bfloat16)
        y = jnp.dot(xp, M_ref[...], preferred_element_type=jnp.float32)
        y = jnp.maximum(y, 0.0).astype(jnp.bfloat16)
        cnn = jnp.dot(y, P_ref[...], preferred_element_type=jnp.float32)
        nw = news_ref[b]
        h = jnp.dot(nw, w1_ref[...], preferred_element_type=jnp.float32)
        h = jnp.maximum(h + b1_ref[0][None, :], 0.0)
        nf = jnp.dot(h, w2_ref[...], preferred_element_type=jnp.float32)
        nf = nf + b2_ref[0][None, :]
        feats += [cnn, nf]
    out_ref[...] = jnp.concatenate(feats, axis=1)


def _fused_features(xpad, news, M, P, npw1, npb1, npw2, npb2):
    return pl.pallas_call(
        _b1_body,
        grid=(NBLK,),
        in_specs=[
            pl.BlockSpec((2, NB, L + 3), lambda i: (0, i, 0)),
            pl.BlockSpec((2, NB, 128), lambda i: (0, i, 0)),
            pl.BlockSpec((L + 3, CNN_C * L), lambda i: (0, 0)),
            pl.BlockSpec((CNN_C * L, CNN_C), lambda i: (0, 0)),
            pl.BlockSpec((128, 64), lambda i: (0, 0)),
            pl.BlockSpec((1, 64), lambda i: (0, 0)),
            pl.BlockSpec((64, 32), lambda i: (0, 0)),
            pl.BlockSpec((1, 32), lambda i: (0, 0)),
        ],
        out_specs=pl.BlockSpec((NB, 2 * GCN_IN), lambda i: (i, 0)),
        out_shape=jax.ShapeDtypeStruct((N, 2 * GCN_IN), jnp.float32),
    )(xpad, news, M, P, npw1, npb1, npw2, npb2)


def _b2_body(f2_ref, W_ref, bb_ref, deg_ref, g0, g1, g2, g3, dinv_ref):
    h = jnp.dot(f2_ref[...], W_ref[...], preferred_element_type=jnp.float32)
    h = h + bb_ref[0][None, :]
    deg = deg_ref[...] + 1.0
    dinv = lax.rsqrt(jnp.maximum(deg, 1.0))
    dinv_ref[...] = dinv
    g = h * dinv
    outs = (g0, g1, g2, g3)
    for c in range(4):
        outs[c][...] = g[:, c * CHW:(c + 1) * CHW]


def _g1_tables(f2, Wbig1, bbig1, deg_col):
    outs = tuple(jax.ShapeDtypeStruct((N, CHW), jnp.float32) for _ in range(4))
    return pl.pallas_call(
        _b2_body,
        grid=(NBLK,),
        in_specs=[
            pl.BlockSpec((NB, 2 * GCN_IN), lambda i: (i, 0)),
            pl.BlockSpec((2 * GCN_IN, 2 * G_HID), lambda i: (0, 0)),
            pl.BlockSpec((1, 2 * G_HID), lambda i: (0, 0)),
            pl.BlockSpec((NB, 1), lambda i: (i, 0)),
        ],
        out_specs=tuple(pl.BlockSpec((NB, CHW), lambda i: (i, 0))
                        for _ in range(4)) + (pl.BlockSpec((NB, 1), lambda i: (i, 0)),),
        out_shape=outs + (jax.ShapeDtypeStruct((N, 1), jnp.float32),),
    )(f2, Wbig1, bbig1, deg_col)


def _make_sc_scatter(nch):
    mesh = plsc.VectorSubcoreMesh(core_axis_name="c", subcore_axis_name="s")

    @functools.partial(
        pl.kernel,
        mesh=mesh,
        out_type=jax.ShapeDtypeStruct((nch, 2, ACC_N, CHW), jnp.float32),
        scratch_types=[
            pltpu.VMEM((TMAX, EB), jnp.int32),
            pltpu.VMEM((TMAX, EB), jnp.int32),
            pltpu.VMEM((NBUF, EB, CHW), jnp.float32),
            pltpu.VMEM_SHARED((ACC_N, CHW), jnp.float32),
            pltpu.SemaphoreType.DMA,
        ],
    )
    def sc_scatter(src_hbm, dst_hbm, zeros_hbm, *rest):
        tables = rest[:nch]
        out = rest[nch]
        src_v, dst_v, gb, acc, sem = rest[nch + 1:]
        c = lax.axis_index("c")
        s = lax.axis_index("s")

        def run(first, nt):
            pltpu.sync_copy(src_hbm.at[s, pl.ds(first, nt)],
                            src_v.at[pl.ds(0, nt)])
            pltpu.sync_copy(dst_hbm.at[s, pl.ds(first, nt)],
                            dst_v.at[pl.ds(0, nt)])
            for ch in range(nch):
                pltpu.sync_copy(zeros_hbm, acc.at[pl.ds(s * SLAB, SLAB)])
                plsc.subcore_barrier()

                for b in range(NBUF):
                    pltpu.async_copy(tables[ch].at[src_v.at[b]], gb.at[b], sem)

                def body(t, carry):
                    b = jnp.bitwise_and(t, NBUF - 1)
                    pltpu.make_async_copy(tables[ch].at[src_v.at[t]],
                                          gb.at[b], sem).wait()
                    pltpu.sync_copy(gb.at[b], acc.at[dst_v.at[t]], add=True)

                    @pl.when(t + NBUF < nt)
                    def _():
                        pltpu.async_copy(tables[ch].at[src_v.at[t + NBUF]],
                                         gb.at[b], sem)
                    return carry

                lax.fori_loop(0, nt, body, 0)
                plsc.subcore_barrier()
                pltpu.sync_copy(acc.at[pl.ds(s * SLAB, SLAB)],
                                out.at[ch, c, pl.ds(s * SLAB, SLAB)])

        @pl.when(c == 0)
        def _():
            run(0, T0)

        @pl.when(c == 1)
        def _():
            run(T0, T1)

    return sc_scatter


_sc_cache = {}


def _sc_scatter(nch, *args):
    if nch not in _sc_cache:
        _sc_cache[nch] = _make_sc_scatter(nch)
    return _sc_cache[nch](*args)


def _c_body(P_ref, g10, g11, g12, g13, dinv_ref, W_ref, bb_ref, o0, o1):
    dinv = dinv_ref[...]
    g1s = (g10, g11, g12, g13)
    xs = []
    for ch in range(4):
        S = P_ref[ch, 0] + P_ref[ch, 1]
        xs.append(jnp.maximum((S + g1s[ch][...]) * dinv, 0.0))
    x1 = jnp.concatenate(xs, axis=1)
    h = jnp.dot(x1, W_ref[...], preferred_element_type=jnp.float32)
    h = h + bb_ref[0][None, :]
    g = h * dinv
    o0[...] = g[:, :CHW]
    o1[...] = g[:, CHW:]


def _layer2_tables(P1, g1s, dinv_col, Wbig2, bbig2):
    outs = tuple(jax.ShapeDtypeStruct((N, CHW), jnp.float32) for _ in range(2))
    return pl.pallas_call(
        _c_body,
        grid=(NBLK,),
        in_specs=[
            pl.BlockSpec((4, 2, NB, CHW), lambda i: (0, 0, i, 0)),
            pl.BlockSpec((NB, CHW), lambda i: (i, 0)),
            pl.BlockSpec((NB, CHW), lambda i: (i, 0)),
            pl.BlockSpec((NB, CHW), lambda i: (i, 0)),
            pl.BlockSpec((NB, CHW), lambda i: (i, 0)),
            pl.BlockSpec((NB, 1), lambda i: (i, 0)),
            pl.BlockSpec((2 * G_HID, 2 * G_OUT), lambda i: (0, 0)),
            pl.BlockSpec((1, 2 * G_OUT), lambda i: (0, 0)),
        ],
        out_specs=tuple(pl.BlockSpec((NB, CHW), lambda i: (i, 0))
                        for _ in range(2)),
        out_shape=outs,
    )(P1, *g1s, dinv_col, Wbig2, bbig2)


def _f_body(P_ref, g20, g21, dinv_ref, W1_ref, b1_ref, W2_ref, b2_ref, o_ref):
    dinv = dinv_ref[...]
    g2s = (g20, g21)
    xs = []
    for ch in range(2):
        S = P_ref[ch, 0] + P_ref[ch, 1]
        xs.append((S + g2s[ch][...]) * dinv)
    x2 = jnp.concatenate(xs, axis=1)
    h = jnp.dot(x2, W1_ref[...], preferred_element_type=jnp.float32)
    h = jnp.maximum(h + b1_ref[0][None, :], 0.0)
    o = jnp.dot(h, W2_ref[...], preferred_element_type=jnp.float32)
    o_ref[...] = o + b2_ref[0][None, :]


def _final_mlp(P2, g2s, dinv_col, Mw1b, mb1b, Mw2b, mb2b):
    return pl.pallas_call(
        _f_body,
        grid=(NBLK,),
        in_specs=[
            pl.BlockSpec((2, 2, NB, CHW), lambda i: (0, 0, i, 0)),
            pl.BlockSpec((NB, CHW), lambda i: (i, 0)),
            pl.BlockSpec((NB, CHW), lambda i: (i, 0)),
            pl.BlockSpec((NB, 1), lambda i: (i, 0)),
            pl.BlockSpec((2 * G_OUT, 2 * G_OUT), lambda i: (0, 0)),
            pl.BlockSpec((1, 2 * G_OUT), lambda i: (0, 0)),
            pl.BlockSpec((2 * G_OUT, 4), lambda i: (0, 0)),
            pl.BlockSpec((1, 4), lambda i: (0, 0)),
        ],
        out_specs=pl.BlockSpec((NB, 4), lambda i: (i, 0)),
        out_shape=jax.ShapeDtypeStruct((N, 4), jnp.float32),
    )(P2, *g2s, dinv_col, Mw1b, mb1b, Mw2b, mb2b)


def _blockdiag2(w):
    r, c = w.shape
    out = jnp.zeros((2 * r, 2 * c), jnp.float32)
    out = out.at[:r, :c].set(w)
    out = out.at[r:, c:].set(w)
    return out


def kernel(price_data_x, edge_index, news_features, conv_w, conv_b,
           npw1, npb1, npw2, npb2, gw1, gb1, gw2, gb2, mw1, mb1, mw2, mb2):
    f32 = jnp.float32
    src = edge_index[0]
    dst = edge_index[1]

    npad = EP - E
    src_pad = jnp.concatenate([src, jnp.zeros((npad,), jnp.int32)])
    dst_pad = jnp.concatenate([dst, jnp.full((npad,), N, jnp.int32)])
    src_r = src_pad.reshape(16, 80, EB)
    dst_r = dst_pad.reshape(16, 80, EB)
    dst_deg = jnp.concatenate([dst, jnp.full((npad,), 10240, jnp.int32)])
    dst_deg2d = dst_deg.reshape(EP // 4096, 1, 4096)
    zeros_slab = jnp.zeros((SLAB, CHW), f32)

    j_i = lax.broadcasted_iota(jnp.int32, (L + 2, CNN_C, L), 0)
    t_i = lax.broadcasted_iota(jnp.int32, (L + 2, CNN_C, L), 2)
    d = j_i - t_i
    M = jnp.zeros((L + 2, CNN_C, L), f32)
    for k in range(3):
        M = M + jnp.where(d == k, conv_w[None, :, 0, k, None], 0.0)
    brep = jnp.broadcast_to(conv_b[:, None], (CNN_C, L)).reshape(1, CNN_C * L)
    M = jnp.concatenate([M.reshape(L + 2, CNN_C * L), brep], axis=0)
    P = jnp.kron(jnp.eye(CNN_C, dtype=f32), jnp.ones((L, 1), f32)) / L

    Wbig1 = _blockdiag2(gw1)
    bbig1 = jnp.concatenate([gb1, gb1]).reshape(1, -1)
    Wbig2 = _blockdiag2(gw2)
    bbig2 = jnp.concatenate([gb2, gb2]).reshape(1, -1)
    Mw1b = _blockdiag2(mw1)
    mb1b = jnp.concatenate([mb1, mb1]).reshape(1, -1)
    Mw2b = _blockdiag2(mw2)
    mb2b = jnp.concatenate([mb2, mb2]).reshape(1, -1)

    xpad = jnp.pad(price_data_x, ((0, 0), (0, 0), (1, 2)))
    xpad = xpad.at[:, :, L + 2].set(1.0)

    deg2d = _deg_hist(dst_deg2d)
    deg_col = deg2d.reshape(10240)[:N].reshape(N, 1)

    f2 = _fused_features(xpad, news_features, M, P,
                         npw1, npb1.reshape(1, -1), npw2, npb2.reshape(1, -1))

    *g1s, dinv_col = _g1_tables(f2, Wbig1, bbig1, deg_col)

    P1 = _sc_scatter(4, src_r, dst_r, zeros_slab, *g1s)

    g2s = _layer2_tables(P1, g1s, dinv_col, Wbig2, bbig2)

    P2 = _sc_scatter(2, src_r, dst_r, zeros_slab, *g2s)

    out4 = _final_mlp(P2, g2s, dinv_col, Mw1b, mb1b, Mw2b, mb2b)
    return out4.reshape(N, 2, 2).transpose(1, 0, 2)


# speedup vs baseline: 1.4258x; 1.0283x over previous
import functools

import jax
import jax.numpy as jnp
from jax import lax
from jax.experimental import pallas as pl
from jax.experimental.pallas import tpu as pltpu
from jax.experimental.pallas import tpu_sc as plsc

N = 10000
E = 160000
L = 64
CNN_C = 64
NEWS_P = 32
GCN_IN = CNN_C + NEWS_P
G_HID = 256
G_OUT = 128

NB = 400
NBLK = N // NB
ACC_N = 10112
SLAB = ACC_N // 16
EB = 128
EP = 16 * 80 * EB
T0 = 64
T1 = 80 - T0
TMAX = max(T0, T1)
CHW = 128
NBUF = 2


def _deg_body(dst_ref, out_ref):
    i = pl.program_id(0)
    d = dst_ref[0, 0, :]
    hi = d >> 7
    lo = d & 127
    oh_hi = (hi[None, :] == lax.broadcasted_iota(jnp.int32, (80, 4096), 0)
             ).astype(jnp.bfloat16)
    oh_lo = (lo[None, :] == lax.broadcasted_iota(jnp.int32, (128, 4096), 0)
             ).astype(jnp.bfloat16)
    contrib = lax.dot_general(oh_hi, oh_lo, (((1,), (1,)), ((), ())),
                              preferred_element_type=jnp.float32)

    @pl.when(i == 0)
    def _():
        out_ref[...] = jnp.zeros_like(out_ref)

    out_ref[...] += contrib


def _deg_hist(dst2d):
    return pl.pallas_call(
        _deg_body,
        grid=(dst2d.shape[0],),
        in_specs=[pl.BlockSpec((1, 1, 4096), lambda i: (i, 0, 0))],
        out_specs=pl.BlockSpec((80, 128), lambda i: (0, 0)),
        out_shape=jax.ShapeDtypeStruct((80, 128), jnp.float32),
    )(dst2d)


def _b1_body(xp_ref, news_ref, M_ref, P_ref, w1_ref, b1_ref, w2_ref,
             b2_ref, out_ref):
    feats = []
    for b in range(2):
        xp = xp_ref[b].astype(jnp.bfloat16)
        y = jnp.dot(xp, M_ref[...], preferred_element_type=jnp.float32)
        y = jnp.maximum(y, 0.0).astype(jnp.bfloat16)
        cnn = jnp.dot(y, P_ref[...], preferred_element_type=jnp.float32)
        nw = news_ref[b]
        h = jnp.dot(nw, w1_ref[...], preferred_element_type=jnp.float32)
        h = jnp.maximum(h + b1_ref[0][None, :], 0.0)
        nf = jnp.dot(h, w2_ref[...], preferred_element_type=jnp.float32)
        nf = nf + b2_ref[0][None, :]
        feats += [cnn, nf]
    out_ref[...] = jnp.concatenate(feats, axis=1)


def _fused_features(xpad, news, M, P, npw1, npb1, npw2, npb2):
    return pl.pallas_call(
        _b1_body,
        grid=(NBLK,),
        in_specs=[
            pl.BlockSpec((2, NB, L + 3), lambda i: (0, i, 0)),
            pl.BlockSpec((2, NB, 128), lambda i: (0, i, 0)),
            pl.BlockSpec((L + 3, CNN_C * L), lambda i: (0, 0)),
            pl.BlockSpec((CNN_C * L, CNN_C), lambda i: (0, 0)),
            pl.BlockSpec((128, 64), lambda i: (0, 0)),
            pl.BlockSpec((1, 64), lambda i: (0, 0)),
            pl.BlockSpec((64, 32), lambda i: (0, 0)),
            pl.BlockSpec((1, 32), lambda i: (0, 0)),
        ],
        out_specs=pl.BlockSpec((NB, 2 * GCN_IN), lambda i: (i, 0)),
        out_shape=jax.ShapeDtypeStruct((N, 2 * GCN_IN), jnp.float32),
    )(xpad, news, M, P, npw1, npb1, npw2, npb2)


def _b2_body(f2_ref, W_ref, bb_ref, deg_ref, g0, g1, g2, g3, dinv_ref):
    h = jnp.dot(f2_ref[...], W_ref[...], preferred_element_type=jnp.float32)
    h = h + bb_ref[0][None, :]
    deg = deg_ref[...] + 1.0
    dinv = lax.rsqrt(jnp.maximum(deg, 1.0))
    dinv_ref[...] = dinv
    g = h * dinv
    outs = (g0, g1, g2, g3)
    for c in range(4):
        outs[c][...] = g[:, c * CHW:(c + 1) * CHW]


def _g1_tables(f2, Wbig1, bbig1, deg_col):
    outs = tuple(jax.ShapeDtypeStruct((N, CHW), jnp.float32) for _ in range(4))
    return pl.pallas_call(
        _b2_body,
        grid=(NBLK,),
        in_specs=[
            pl.BlockSpec((NB, 2 * GCN_IN), lambda i: (i, 0)),
            pl.BlockSpec((2 * GCN_IN, 2 * G_HID), lambda i: (0, 0)),
            pl.BlockSpec((1, 2 * G_HID), lambda i: (0, 0)),
            pl.BlockSpec((NB, 1), lambda i: (i, 0)),
        ],
        out_specs=tuple(pl.BlockSpec((NB, CHW), lambda i: (i, 0))
                        for _ in range(4)) + (pl.BlockSpec((NB, 1), lambda i: (i, 0)),),
        out_shape=outs + (jax.ShapeDtypeStruct((N, 1), jnp.float32),),
    )(f2, Wbig1, bbig1, deg_col)


def _make_sc_scatter(nch):
    mesh = plsc.VectorSubcoreMesh(core_axis_name="c", subcore_axis_name="s")

    @functools.partial(
        pl.kernel,
        mesh=mesh,
        out_type=jax.ShapeDtypeStruct((nch, 2, ACC_N, CHW), jnp.float32),
        scratch_types=[
            pltpu.VMEM((TMAX, EB), jnp.int32),
            pltpu.VMEM((TMAX, EB), jnp.int32),
            pltpu.VMEM((NBUF, EB, CHW), jnp.float32),
            pltpu.VMEM_SHARED((ACC_N, CHW), jnp.float32),
            pltpu.SemaphoreType.DMA,
            pltpu.SemaphoreType.DMA,
        ],
    )
    def sc_scatter(src_hbm, dst_hbm, zeros_hbm, *rest):
        tables = rest[:nch]
        out = rest[nch]
        src_v, dst_v, gb, acc, sem, ssem = rest[nch + 1:]
        c = lax.axis_index("c")
        s = lax.axis_index("s")

        def run(first, nt):
            pltpu.sync_copy(src_hbm.at[s, pl.ds(first, nt)],
                            src_v.at[pl.ds(0, nt)])
            pltpu.sync_copy(dst_hbm.at[s, pl.ds(first, nt)],
                            dst_v.at[pl.ds(0, nt)])
            for ch in range(nch):
                pltpu.sync_copy(zeros_hbm, acc.at[pl.ds(s * SLAB, SLAB)])
                plsc.subcore_barrier()

                for b in range(NBUF):
                    pltpu.async_copy(tables[ch].at[src_v.at[b]], gb.at[b], sem)

                def body(t, carry):
                    b = jnp.bitwise_and(t, NBUF - 1)
                    pltpu.make_async_copy(tables[ch].at[src_v.at[t]],
                                          gb.at[b], sem).wait()
                    pltpu.async_copy(gb.at[b], acc.at[dst_v.at[t]], ssem,
                                     add=True)

                    @pl.when(t + NBUF < nt)
                    def _():
                        pltpu.make_async_copy(gb.at[b], acc.at[dst_v.at[t]],
                                              ssem).wait()
                        pltpu.async_copy(tables[ch].at[src_v.at[t + NBUF]],
                                         gb.at[b], sem)
                    return carry

                lax.fori_loop(0, nt, body, 0)
                for b in range(NBUF):
                    pltpu.make_async_copy(gb.at[b], acc.at[dst_v.at[b]],
                                          ssem).wait()
                plsc.subcore_barrier()
                pltpu.sync_copy(acc.at[pl.ds(s * SLAB, SLAB)],
                                out.at[ch, c, pl.ds(s * SLAB, SLAB)])

        @pl.when(c == 0)
        def _():
            run(0, T0)

        @pl.when(c == 1)
        def _():
            run(T0, T1)

    return sc_scatter


_sc_cache = {}


def _sc_scatter(nch, *args):
    if nch not in _sc_cache:
        _sc_cache[nch] = _make_sc_scatter(nch)
    return _sc_cache[nch](*args)


def _c_body(P_ref, g10, g11, g12, g13, dinv_ref, W_ref, bb_ref, o0, o1):
    dinv = dinv_ref[...]
    g1s = (g10, g11, g12, g13)
    xs = []
    for ch in range(4):
        S = P_ref[ch, 0] + P_ref[ch, 1]
        xs.append(jnp.maximum((S + g1s[ch][...]) * dinv, 0.0))
    x1 = jnp.concatenate(xs, axis=1)
    h = jnp.dot(x1, W_ref[...], preferred_element_type=jnp.float32)
    h = h + bb_ref[0][None, :]
    g = h * dinv
    o0[...] = g[:, :CHW]
    o1[...] = g[:, CHW:]


def _layer2_tables(P1, g1s, dinv_col, Wbig2, bbig2):
    outs = tuple(jax.ShapeDtypeStruct((N, CHW), jnp.float32) for _ in range(2))
    return pl.pallas_call(
        _c_body,
        grid=(NBLK,),
        in_specs=[
            pl.BlockSpec((4, 2, NB, CHW), lambda i: (0, 0, i, 0)),
            pl.BlockSpec((NB, CHW), lambda i: (i, 0)),
            pl.BlockSpec((NB, CHW), lambda i: (i, 0)),
            pl.BlockSpec((NB, CHW), lambda i: (i, 0)),
            pl.BlockSpec((NB, CHW), lambda i: (i, 0)),
            pl.BlockSpec((NB, 1), lambda i: (i, 0)),
            pl.BlockSpec((2 * G_HID, 2 * G_OUT), lambda i: (0, 0)),
            pl.BlockSpec((1, 2 * G_OUT), lambda i: (0, 0)),
        ],
        out_specs=tuple(pl.BlockSpec((NB, CHW), lambda i: (i, 0))
                        for _ in range(2)),
        out_shape=outs,
    )(P1, *g1s, dinv_col, Wbig2, bbig2)


def _f_body(P_ref, g20, g21, dinv_ref, W1_ref, b1_ref, W2_ref, b2_ref, o_ref):
    dinv = dinv_ref[...]
    g2s = (g20, g21)
    xs = []
    for ch in range(2):
        S = P_ref[ch, 0] + P_ref[ch, 1]
        xs.append((S + g2s[ch][...]) * dinv)
    x2 = jnp.concatenate(xs, axis=1)
    h = jnp.dot(x2, W1_ref[...], preferred_element_type=jnp.float32)
    h = jnp.maximum(h + b1_ref[0][None, :], 0.0)
    o = jnp.dot(h, W2_ref[...], preferred_element_type=jnp.float32)
    o_ref[...] = o + b2_ref[0][None, :]


def _final_mlp(P2, g2s, dinv_col, Mw1b, mb1b, Mw2b, mb2b):
    return pl.pallas_call(
        _f_body,
        grid=(NBLK,),
        in_specs=[
            pl.BlockSpec((2, 2, NB, CHW), lambda i: (0, 0, i, 0)),
            pl.BlockSpec((NB, CHW), lambda i: (i, 0)),
            pl.BlockSpec((NB, CHW), lambda i: (i, 0)),
            pl.BlockSpec((NB, 1), lambda i: (i, 0)),
            pl.BlockSpec((2 * G_OUT, 2 * G_OUT), lambda i: (0, 0)),
            pl.BlockSpec((1, 2 * G_OUT), lambda i: (0, 0)),
            pl.BlockSpec((2 * G_OUT, 4), lambda i: (0, 0)),
            pl.BlockSpec((1, 4), lambda i: (0, 0)),
        ],
        out_specs=pl.BlockSpec((NB, 4), lambda i: (i, 0)),
        out_shape=jax.ShapeDtypeStruct((N, 4), jnp.float32),
    )(P2, *g2s, dinv_col, Mw1b, mb1b, Mw2b, mb2b)


def _blockdiag2(w):
    r, c = w.shape
    out = jnp.zeros((2 * r, 2 * c), jnp.float32)
    out = out.at[:r, :c].set(w)
    out = out.at[r:, c:].set(w)
    return out


def kernel(price_data_x, edge_index, news_features, conv_w, conv_b,
           npw1, npb1, npw2, npb2, gw1, gb1, gw2, gb2, mw1, mb1, mw2, mb2):
    f32 = jnp.float32
    src = edge_index[0]
    dst = edge_index[1]

    npad = EP - E
    src_pad = jnp.concatenate([src, jnp.zeros((npad,), jnp.int32)])
    dst_pad = jnp.concatenate([dst, jnp.full((npad,), N, jnp.int32)])
    src_r = src_pad.reshape(16, 80, EB)
    dst_r = dst_pad.reshape(16, 80, EB)
    dst_deg = jnp.concatenate([dst, jnp.full((npad,), 10240, jnp.int32)])
    dst_deg2d = dst_deg.reshape(EP // 4096, 1, 4096)
    zeros_slab = jnp.zeros((SLAB, CHW), f32)

    j_i = lax.broadcasted_iota(jnp.int32, (L + 2, CNN_C, L), 0)
    t_i = lax.broadcasted_iota(jnp.int32, (L + 2, CNN_C, L), 2)
    d = j_i - t_i
    M = jnp.zeros((L + 2, CNN_C, L), f32)
    for k in range(3):
        M = M + jnp.where(d == k, conv_w[None, :, 0, k, None], 0.0)
    brep = jnp.broadcast_to(conv_b[:, None], (CNN_C, L)).reshape(1, CNN_C * L)
    M = jnp.concatenate([M.reshape(L + 2, CNN_C * L), brep], axis=0)
    P = jnp.kron(jnp.eye(CNN_C, dtype=f32), jnp.ones((L, 1), f32)) / L

    Wbig1 = _blockdiag2(gw1)
    bbig1 = jnp.concatenate([gb1, gb1]).reshape(1, -1)
    Wbig2 = _blockdiag2(gw2)
    bbig2 = jnp.concatenate([gb2, gb2]).reshape(1, -1)
    Mw1b = _blockdiag2(mw1)
    mb1b = jnp.concatenate([mb1, mb1]).reshape(1, -1)
    Mw2b = _blockdiag2(mw2)
    mb2b = jnp.concatenate([mb2, mb2]).reshape(1, -1)

    xpad = jnp.pad(price_data_x, ((0, 0), (0, 0), (1, 2)))
    xpad = xpad.at[:, :, L + 2].set(1.0)

    deg2d = _deg_hist(dst_deg2d)
    deg_col = deg2d.reshape(10240)[:N].reshape(N, 1)

    f2 = _fused_features(xpad, news_features, M, P,
                         npw1, npb1.reshape(1, -1), npw2, npb2.reshape(1, -1))

    *g1s, dinv_col = _g1_tables(f2, Wbig1, bbig1, deg_col)

    P1 = _sc_scatter(4, src_r, dst_r, zeros_slab, *g1s)

    g2s = _layer2_tables(P1, g1s, dinv_col, Wbig2, bbig2)

    P2 = _sc_scatter(2, src_r, dst_r, zeros_slab, *g2s)

    out4 = _final_mlp(P2, g2s, dinv_col, Mw1b, mb1b, Mw2b, mb2b)
    return out4.reshape(N, 2, 2).transpose(1, 0, 2)
